# Initial kernel scaffold; baseline (speedup 1.0000x reference)
import jax
import jax.numpy as jnp
from jax.experimental import pallas as pl


def kernel(x):
    raise NotImplementedError("write your pallas kernel here")


# speedup vs baseline: 98.4323x; 98.4323x over previous
import jax
import jax.numpy as jnp
from jax import lax
from jax.experimental import pallas as pl
from jax.experimental.pallas import tpu as pltpu
from jax.experimental.pallas import tpu_sc as plsc

_RADIUS2 = 3.4 * 3.4
_NSAMPLE = 5
_B = 8
_N = 2048
_L = 16
_NC = 2
_NS = 16
_NW = _NC * _NS
_WPB = _NW // _B
_QPW = _N // _WPB
_GROUPS = _QPW // _L


def _ball_query_body(xs_hbm, ys_hbm, zs_hbm, out_hbm, xs_v, ys_v, zs_v,
                     out_v):
    c = lax.axis_index("c")
    s = lax.axis_index("s")
    wid = s * _NC + c
    b = wid // _WPB
    qoff = (wid % _WPB) * _QPW

    pltpu.sync_copy(xs_hbm.at[pl.ds(b * _N, _N)], xs_v)
    pltpu.sync_copy(ys_hbm.at[pl.ds(b * _N, _N)], ys_v)
    pltpu.sync_copy(zs_hbm.at[pl.ds(b * _N, _N)], zs_v)

    lanes = lax.iota(jnp.int32, _L)
    zero = jnp.zeros((_L,), jnp.int32)

    def group(g, carry_none):
        qbase = qoff + g * _L
        qx = xs_v[pl.ds(qbase, _L)]
        qy = ys_v[pl.ds(qbase, _L)]
        qz = zs_v[pl.ds(qbase, _L)]

        def cond(carry):
            return carry[0]

        def body(carry):
            _, j, cnt, i0, i1, i2, i3, i4 = carry
            jv = jnp.full((_L,), j, dtype=jnp.int32)
            px = plsc.load_gather(xs_v, [jv])
            py = plsc.load_gather(ys_v, [jv])
            pz = plsc.load_gather(zs_v, [jv])
            dx = qx - px
            dy = qy - py
            dz = qz - pz
            d2 = dx * dx + dy * dy + dz * dz
            m = d2 < _RADIUS2
            i0 = jnp.where(m & (cnt == 0), jv, i0)
            i1 = jnp.where(m & (cnt == 1), jv, i1)
            i2 = jnp.where(m & (cnt == 2), jv, i2)
            i3 = jnp.where(m & (cnt == 3), jv, i3)
            i4 = jnp.where(m & (cnt == 4), jv, i4)
            cnt = cnt + m.astype(jnp.int32)
            jn = j + 1
            cont = jnp.logical_and(
                jn < _N, jnp.logical_not(jnp.all(cnt >= _NSAMPLE))
            )
            return (cont, jn, cnt, i0, i1, i2, i3, i4)

        init = (jnp.bool_(True), jnp.int32(0), zero, zero, zero, zero, zero,
                zero)
        _, _, cnt, i0, i1, i2, i3, i4 = lax.while_loop(cond, body, init)

        o1 = jnp.where(cnt > 1, i1, i0)
        o2 = jnp.where(cnt > 2, i2, i0)
        o3 = jnp.where(cnt > 3, i3, i0)
        o4 = jnp.where(cnt > 4, i4, i0)
        rows = (g * _L + lanes) * _NSAMPLE
        plsc.store_scatter(out_v, [rows], i0)
        plsc.store_scatter(out_v, [rows + 1], o1)
        plsc.store_scatter(out_v, [rows + 2], o2)
        plsc.store_scatter(out_v, [rows + 3], o3)
        plsc.store_scatter(out_v, [rows + 4], o4)
        return carry_none

    lax.fori_loop(0, _GROUPS, group, 0)

    chunk = _QPW * _NSAMPLE
    pltpu.sync_copy(out_v, out_hbm.at[pl.ds(wid * chunk, chunk)])


def kernel(x):
    xs = x[:, :, 0].reshape(-1)
    ys = x[:, :, 1].reshape(-1)
    zs = x[:, :, 2].reshape(-1)
    mesh = plsc.VectorSubcoreMesh(core_axis_name="c", subcore_axis_name="s")
    out = pl.kernel(
        _ball_query_body,
        out_type=jax.ShapeDtypeStruct((_B * _N * _NSAMPLE,), jnp.int32),
        mesh=mesh,
        compiler_params=pltpu.CompilerParams(needs_layout_passes=False),
        scratch_types=[
            pltpu.VMEM((_N,), jnp.float32),
            pltpu.VMEM((_N,), jnp.float32),
            pltpu.VMEM((_N,), jnp.float32),
            pltpu.VMEM((_QPW * _NSAMPLE,), jnp.int32),
        ],
    )(xs, ys, zs)
    return out.reshape(_B, _N, _NSAMPLE)
